# baseline (speedup 1.0000x reference)
import functools

import jax
import jax.numpy as jnp
from jax import lax
from jax.experimental import pallas as pl
from jax.experimental.pallas import tpu as pltpu
from jax.experimental.pallas import tpu_sc as plsc

_HIDDEN = 64
_GAMMA = 12.0
_BATCH = 16384
_NENT = 100000
_PACK = 4
_NC = 2
_NS = 16
_NW = _NC * _NS
_RPW = _BATCH // _NW
_CHUNK = 128
_NCHUNK = _RPW // _CHUNK
_LANES = 16
_NGRP = _CHUNK // _LANES
_ROWW = 128
_SUBW = _ROWW // _PACK


@functools.partial(
    pl.kernel,
    out_type=jax.ShapeDtypeStruct((_BATCH,), jnp.float32),
    mesh=plsc.VectorSubcoreMesh(core_axis_name="c", subcore_axis_name="s"),
    compiler_params=pltpu.CompilerParams(needs_layout_passes=False),
    scratch_types=[
        pltpu.VMEM((_RPW,), jnp.int32),
        pltpu.VMEM((_RPW,), jnp.int32),
        pltpu.VMEM((_RPW,), jnp.int32),
        pltpu.VMEM((2, _CHUNK, _ROWW), jnp.int32),
        pltpu.VMEM((2, _CHUNK, _ROWW), jnp.int32),
        pltpu.VMEM((2, _CHUNK, _ROWW), jnp.int32),
        pltpu.VMEM((_RPW,), jnp.float32),
        pltpu.SemaphoreType.DMA,
        pltpu.SemaphoreType.DMA,
    ],
)
def _transe_score(hidx_hbm, ridx_hbm, tidx_hbm, ent_hbm, rel_hbm, out_hbm,
                  hidx_v, ridx_v, tidx_v, hrows, rrows, trows, out_v,
                  sem0, sem1):
    sems = (sem0, sem1)
    wid = lax.axis_index("s") * _NC + lax.axis_index("c")
    base = wid * _RPW
    lane = lax.iota(jnp.int32, _LANES)

    pltpu.sync_copy(hidx_hbm.at[pl.ds(base, _RPW)], hidx_v)
    pltpu.sync_copy(ridx_hbm.at[pl.ds(base, _RPW)], ridx_v)
    pltpu.sync_copy(tidx_hbm.at[pl.ds(base, _RPW)], tidx_v)

    def issue_chunk(j):
        b = j % 2
        sem = sems[b]

        def grp(g, carry):
            i0 = j * _CHUNK + g * _LANES
            hvec = hidx_v[pl.ds(i0, _LANES)]
            rvec = ridx_v[pl.ds(i0, _LANES)]
            tvec = tidx_v[pl.ds(i0, _LANES)]
            d0 = g * _LANES
            for s in range(_LANES):
                pltpu.async_copy(
                    ent_hbm.at[pl.ds(hvec[s] >> 2, 1)],
                    hrows.at[b, pl.ds(d0 + s, 1)], sem)
                pltpu.async_copy(
                    rel_hbm.at[pl.ds(rvec[s] >> 2, 1)],
                    rrows.at[b, pl.ds(d0 + s, 1)], sem)
                pltpu.async_copy(
                    ent_hbm.at[pl.ds(tvec[s] >> 2, 1)],
                    trows.at[b, pl.ds(d0 + s, 1)], sem)
            return carry

        lax.fori_loop(0, _NGRP, grp, 0)

    def drain_chunk(j):
        b = j % 2
        sem = sems[b]
        pltpu.make_async_copy(
            ent_hbm.at[pl.ds(0, _CHUNK)], hrows.at[b], sem).wait()
        pltpu.make_async_copy(
            ent_hbm.at[pl.ds(0, _CHUNK)], rrows.at[b], sem).wait()
        pltpu.make_async_copy(
            ent_hbm.at[pl.ds(0, _CHUNK)], trows.at[b], sem).wait()

    def compute_chunk(j):
        b = j % 2
        h2, r2, t2 = hrows.at[b], rrows.at[b], trows.at[b]

        def grp(g, carry):
            g0 = g * _LANES
            i0 = j * _CHUNK + g0
            hvec = hidx_v[pl.ds(i0, _LANES)]
            rvec = ridx_v[pl.ds(i0, _LANES)]
            tvec = tidx_v[pl.ds(i0, _LANES)]
            outacc = jnp.zeros((_LANES,), jnp.float32)
            for s in range(_LANES):
                i = g0 + s
                ho = (hvec[s] & (_PACK - 1)) * _SUBW
                ro = (rvec[s] & (_PACK - 1)) * _SUBW
                to = (tvec[s] & (_PACK - 1)) * _SUBW
                acc = jnp.zeros((_LANES,), jnp.float32)
                for k in range(2):
                    hb = plsc.bitcast(
                        h2[i, pl.ds(ho + k * _LANES, _LANES)], jnp.bfloat16)
                    rb = plsc.bitcast(
                        r2[i, pl.ds(ro + k * _LANES, _LANES)], jnp.bfloat16)
                    tb = plsc.bitcast(
                        t2[i, pl.ds(to + k * _LANES, _LANES)], jnp.bfloat16)
                    a = jnp.abs(hb + rb - tb)
                    a0, a1 = plsc.unpack(
                        a, format=plsc.PackFormat.INTERLEAVED)
                    acc = acc + a0 + a1
                tot = jnp.full((_LANES,), _GAMMA - jnp.sum(acc), jnp.float32)
                outacc = jnp.where(lane == s, tot, outacc)
            out_v[pl.ds(j * _CHUNK + g0, _LANES)] = outacc
            return carry

        lax.fori_loop(0, _NGRP, grp, 0)

    issue_chunk(0)
    for j in range(_NCHUNK):
        if j + 1 < _NCHUNK:
            issue_chunk(j + 1)
        drain_chunk(j)
        compute_chunk(j)

    pltpu.sync_copy(out_v, out_hbm.at[pl.ds(base, _RPW)])


def _pack_table(table):
    t16 = table.astype(jnp.bfloat16)
    return jax.lax.bitcast_convert_type(
        t16.reshape(_NENT // _PACK, _ROWW, 2), jnp.int32)


def kernel(sample, entity_embedding, relation_embedding):
    ent_pk = _pack_table(entity_embedding)
    rel_pk = _pack_table(relation_embedding)
    score = _transe_score(sample[:, 0], sample[:, 1], sample[:, 2],
                          ent_pk, rel_pk)
    return score.reshape(_BATCH, 1)


# speedup vs baseline: 45.1810x; 45.1810x over previous
import functools

import jax
import jax.numpy as jnp
from jax import lax
from jax.experimental import pallas as pl
from jax.experimental.pallas import tpu as pltpu
from jax.experimental.pallas import tpu_sc as plsc

_HIDDEN = 64
_GAMMA = 12.0
_BATCH = 16384
_NC = 2
_NS = 16
_NW = _NC * _NS
_RPW = _BATCH // _NW
_CHUNK = 128
_NCHUNK = _RPW // _CHUNK
_LANES = 16
_NGRP = _CHUNK // _LANES

_SC_PARAMS = pltpu.CompilerParams(needs_layout_passes=False)
_MESH = plsc.VectorSubcoreMesh(core_axis_name="c", subcore_axis_name="s")


def _wid_base():
    wid = lax.axis_index("s") * _NC + lax.axis_index("c")
    return wid * _RPW


@functools.partial(
    pl.kernel,
    out_type=jax.ShapeDtypeStruct((_BATCH, _HIDDEN), jnp.float32),
    mesh=_MESH,
    compiler_params=_SC_PARAMS,
    scratch_types=[
        pltpu.VMEM((_RPW,), jnp.int32),
        pltpu.VMEM((_RPW,), jnp.int32),
        pltpu.VMEM((2, _CHUNK, _HIDDEN), jnp.float32),
        pltpu.VMEM((2, _CHUNK, _HIDDEN), jnp.float32),
        pltpu.VMEM((2, _CHUNK, _HIDDEN), jnp.float32),
        pltpu.SemaphoreType.DMA,
        pltpu.SemaphoreType.DMA,
    ],
)
def _gather_ht(hidx_hbm, tidx_hbm, ent_hbm, ht_hbm,
               hidx_v, tidx_v, hrows, trows, htout, sem0, sem1):
    sems = (sem0, sem1)
    base = _wid_base()

    pltpu.sync_copy(hidx_hbm.at[pl.ds(base, _RPW)], hidx_v)
    pltpu.sync_copy(tidx_hbm.at[pl.ds(base, _RPW)], tidx_v)

    def issue_chunk(j):
        b = j % 2
        sem = sems[b]

        def grp(g, carry):
            i0 = j * _CHUNK + g * _LANES
            hvec = hidx_v[pl.ds(i0, _LANES)]
            tvec = tidx_v[pl.ds(i0, _LANES)]
            d0 = g * _LANES
            for s in range(_LANES):
                pltpu.async_copy(
                    ent_hbm.at[pl.ds(hvec[s], 1)],
                    hrows.at[b, pl.ds(d0 + s, 1)], sem)
                pltpu.async_copy(
                    ent_hbm.at[pl.ds(tvec[s], 1)],
                    trows.at[b, pl.ds(d0 + s, 1)], sem)
            return carry

        lax.fori_loop(0, _NGRP, grp, 0)

    def drain_chunk(j):
        b = j % 2
        sem = sems[b]
        pltpu.make_async_copy(
            ent_hbm.at[pl.ds(0, _CHUNK)], hrows.at[b], sem).wait()
        pltpu.make_async_copy(
            ent_hbm.at[pl.ds(0, _CHUNK)], trows.at[b], sem).wait()

    def compute_chunk(j):
        b = j % 2
        h2, t2 = hrows.at[b], trows.at[b]

        def samp(i, carry):
            for k in range(_HIDDEN // _LANES):
                sl = pl.ds(k * _LANES, _LANES)
                htout[b, i, sl] = h2[i, sl] - t2[i, sl]
            return carry

        lax.fori_loop(0, _CHUNK, samp, 0)
        pltpu.sync_copy(
            htout.at[b], ht_hbm.at[pl.ds(base + j * _CHUNK, _CHUNK)])

    issue_chunk(0)
    for j in range(_NCHUNK):
        if j + 1 < _NCHUNK:
            issue_chunk(j + 1)
        drain_chunk(j)
        compute_chunk(j)


@functools.partial(
    pl.kernel,
    out_type=jax.ShapeDtypeStruct((_BATCH,), jnp.float32),
    mesh=_MESH,
    compiler_params=_SC_PARAMS,
    scratch_types=[
        pltpu.VMEM((_RPW,), jnp.int32),
        pltpu.VMEM((2, _CHUNK, _HIDDEN), jnp.float32),
        pltpu.VMEM((2, _CHUNK, _HIDDEN), jnp.float32),
        pltpu.VMEM((_RPW,), jnp.float32),
        pltpu.SemaphoreType.DMA,
        pltpu.SemaphoreType.DMA,
    ],
)
def _score_r(ridx_hbm, rel_hbm, ht_hbm, out_hbm,
             ridx_v, rrows, htc, out_v, sem0, sem1):
    sems = (sem0, sem1)
    base = _wid_base()
    lane = lax.iota(jnp.int32, _LANES)

    pltpu.sync_copy(ridx_hbm.at[pl.ds(base, _RPW)], ridx_v)

    def issue_chunk(j):
        b = j % 2
        sem = sems[b]
        pltpu.async_copy(
            ht_hbm.at[pl.ds(base + j * _CHUNK, _CHUNK)], htc.at[b], sem)

        def grp(g, carry):
            i0 = j * _CHUNK + g * _LANES
            rvec = ridx_v[pl.ds(i0, _LANES)]
            d0 = g * _LANES
            for s in range(_LANES):
                pltpu.async_copy(
                    rel_hbm.at[pl.ds(rvec[s], 1)],
                    rrows.at[b, pl.ds(d0 + s, 1)], sem)
            return carry

        lax.fori_loop(0, _NGRP, grp, 0)

    def drain_chunk(j):
        b = j % 2
        sem = sems[b]
        pltpu.make_async_copy(
            rel_hbm.at[pl.ds(0, _CHUNK)], rrows.at[b], sem).wait()
        pltpu.make_async_copy(
            ht_hbm.at[pl.ds(0, _CHUNK)], htc.at[b], sem).wait()

    def compute_chunk(j):
        b = j % 2
        r2, d1 = rrows.at[b], htc.at[b]

        def grp(g, carry):
            g0 = g * _LANES
            outacc = jnp.zeros((_LANES,), jnp.float32)
            for s in range(_LANES):
                i = g0 + s
                acc = jnp.zeros((_LANES,), jnp.float32)
                for k in range(_HIDDEN // _LANES):
                    sl = pl.ds(k * _LANES, _LANES)
                    acc = acc + jnp.abs(d1[i, sl] + r2[i, sl])
                tot = jnp.full((_LANES,), _GAMMA - jnp.sum(acc), jnp.float32)
                outacc = jnp.where(lane == s, tot, outacc)
            out_v[pl.ds(j * _CHUNK + g0, _LANES)] = outacc
            return carry

        lax.fori_loop(0, _NGRP, grp, 0)

    issue_chunk(0)
    for j in range(_NCHUNK):
        if j + 1 < _NCHUNK:
            issue_chunk(j + 1)
        drain_chunk(j)
        compute_chunk(j)

    pltpu.sync_copy(out_v, out_hbm.at[pl.ds(base, _RPW)])


def kernel(sample, entity_embedding, relation_embedding):
    ht = _gather_ht(sample[:, 0], sample[:, 2], entity_embedding)
    score = _score_r(sample[:, 1], relation_embedding, ht)
    return score.reshape(_BATCH, 1)
